# baseline (speedup 1.0000x reference)
import functools

import jax
import jax.numpy as jnp
from jax import lax
from jax.experimental import pallas as pl
from jax.experimental.pallas import tpu as pltpu
from jax.experimental.pallas import tpu_sc as plsc

NC = 2
NS = 16
CHUNK = 256
WSC = 80
HALF = 64


SEG = 9


NBUF = 3


def _sc_scatter_body(cpt, rpt, xcat_hbm, src_hbm, dst_hbm, zeros_hbm, out_hbm,
                     src_v, dst_v, rows_v, acc_sh, *sems):
    cid = lax.axis_index("c")
    sid = lax.axis_index("s")
    sem_g = sems[:NBUF]
    sem_s = sems[NBUF:]

    def start_gather(j, b):
        pltpu.async_copy(xcat_hbm.at[src_v.at[j]], rows_v.at[b], sem_g[b])

    def wait_gather(j, b):
        pltpu.make_async_copy(xcat_hbm.at[src_v.at[j]], rows_v.at[b],
                              sem_g[b]).wait()

    def start_scatter(j, b):
        pltpu.async_copy(rows_v.at[b], acc_sh.at[dst_v.at[j]], sem_s[b],
                         add=True)

    def wait_scatter(j, b):
        pltpu.make_async_copy(rows_v.at[b], acc_sh.at[dst_v.at[j]],
                              sem_s[b]).wait()

    pltpu.sync_copy(zeros_hbm, acc_sh.at[pl.ds(sid * rpt, rpt)])
    plsc.subcore_barrier()

    def seg_body(s, carry):
        pltpu.sync_copy(
            src_hbm.at[pl.ds((cid * NS + sid) * cpt + s * SEG, SEG)], src_v)
        pltpu.sync_copy(dst_hbm.at[pl.ds(sid * cpt + s * SEG, SEG)], dst_v)

        start_gather(0, 0)
        start_gather(1, 1)

        def body(j0, c2):
            for u in range(NBUF):
                j = NBUF * j0 + u
                b = u
                wait_gather(j, b)
                start_scatter(j, b)

                @pl.when(j < SEG - 2)
                def _():
                    @pl.when(j >= 1)
                    def _():
                        wait_scatter(j - 1, (u - 1) % NBUF)

                    start_gather(j + 2, (u + 2) % NBUF)
            return c2

        lax.fori_loop(0, SEG // NBUF, body, 0)
        wait_scatter(SEG - 3, (SEG - 3) % NBUF)
        wait_scatter(SEG - 2, (SEG - 2) % NBUF)
        wait_scatter(SEG - 1, (SEG - 1) % NBUF)
        return carry

    lax.fori_loop(0, cpt // SEG, seg_body, 0)
    plsc.subcore_barrier()
    pltpu.sync_copy(acc_sh.at[pl.ds(sid * rpt, rpt)],
                    out_hbm.at[cid, pl.ds(sid * rpt, rpt)])


def _tc_matmul_body(h, acc_ref, x_ref, w_ref, o_ref):
    a0 = acc_ref[0]
    a1 = acc_ref[1]
    g_lo = a0[:, :HALF]
    deg = a0[:, HALF:HALF + 1]
    g_hi = a1[:, :HALF]
    o_ref[...] = (
        jnp.dot(g_lo, w_ref[:HALF], preferred_element_type=jnp.float32)
        + jnp.dot(g_hi, w_ref[HALF:h], preferred_element_type=jnp.float32)
        + jnp.dot(deg * x_ref[...], w_ref[h:], preferred_element_type=jnp.float32)
    )


def kernel(x, edge_index, W):
    n, h = x.shape
    e = edge_index.shape[1]

    rpt = (-(-(n + 1) // NS) + 7) // 8 * 8
    n_acc = rpt * NS

    cpt = -(-(-(-(2 * e) // (NS * CHUNK))) // SEG) * SEG
    pad_e = NS * cpt * CHUNK - 2 * e
    src = edge_index[0].astype(jnp.int32)
    dst = edge_index[1].astype(jnp.int32)
    pad = jnp.full((pad_e,), n, jnp.int32)
    src_all = jnp.concatenate([src, dst, pad]).reshape(NS * cpt, CHUNK)
    dst_all = jnp.concatenate([dst, src, pad]).reshape(NS * cpt, CHUNK)
    src_both = jnp.concatenate([src_all, src_all + n_acc], axis=0)

    xcat = (jnp.zeros((2 * n_acc, WSC), jnp.float32)
            .at[:n, :HALF].set(x[:, :HALF])
            .at[:n, HALF].set(1.0)
            .at[n_acc:n_acc + n, :HALF].set(x[:, HALF:]))
    zeros_blk = jnp.zeros((rpt, WSC), jnp.float32)

    sc_scatter = pl.kernel(
        functools.partial(_sc_scatter_body, cpt, rpt),
        out_type=jax.ShapeDtypeStruct((NC, n_acc, WSC), jnp.float32),
        mesh=plsc.VectorSubcoreMesh(core_axis_name="c", subcore_axis_name="s",
                                    num_cores=NC, num_subcores=NS),
        scratch_types=[
            pltpu.VMEM((SEG, CHUNK), jnp.int32),
            pltpu.VMEM((SEG, CHUNK), jnp.int32),
            pltpu.VMEM((NBUF, CHUNK, WSC), jnp.float32),
            pltpu.VMEM_SHARED((n_acc, WSC), jnp.float32),
        ] + [pltpu.SemaphoreType.DMA] * (2 * NBUF),
        compiler_params=pltpu.CompilerParams(use_tc_tiling_on_sc=False),
    )
    acc = sc_scatter(xcat, src_both, dst_all, zeros_blk)

    x_pad = jnp.zeros((n_acc, h), jnp.float32).at[:n].set(x)
    out = pl.pallas_call(
        functools.partial(_tc_matmul_body, h),
        grid=(NS,),
        in_specs=[
            pl.BlockSpec((NC, rpt, WSC), lambda i: (0, i, 0)),
            pl.BlockSpec((rpt, h), lambda i: (i, 0)),
            pl.BlockSpec((2 * h, h), lambda i: (0, 0)),
        ],
        out_specs=pl.BlockSpec((rpt, h), lambda i: (i, 0)),
        out_shape=jax.ShapeDtypeStruct((n_acc, h), jnp.float32),
    )(acc, x_pad, W)
    return out[:n]


# speedup vs baseline: 1.1334x; 1.1334x over previous
import functools

import jax
import jax.numpy as jnp
from jax import lax
from jax.experimental import pallas as pl
from jax.experimental.pallas import tpu as pltpu
from jax.experimental.pallas import tpu_sc as plsc

NC = 2
NS = 16
CHUNK = 128
WSC = 80
HALF = 64


SEG = 36


NBUF = 6


def _sc_scatter_body(cpt, rpt, xcat_hbm, src_hbm, dst_hbm, zeros_hbm, out_hbm,
                     src_v, dst_v, rows_v, acc_sh, *sems):
    cid = lax.axis_index("c")
    sid = lax.axis_index("s")
    sem_g = sems[:NBUF]
    sem_s = sems[NBUF:]

    def start_gather(j, b):
        pltpu.async_copy(xcat_hbm.at[src_v.at[j]], rows_v.at[b], sem_g[b])

    def wait_gather(j, b):
        pltpu.make_async_copy(xcat_hbm.at[src_v.at[j]], rows_v.at[b],
                              sem_g[b]).wait()

    def start_scatter(j, b):
        pltpu.async_copy(rows_v.at[b], acc_sh.at[dst_v.at[j]], sem_s[b],
                         add=True)

    def wait_scatter(j, b):
        pltpu.make_async_copy(rows_v.at[b], acc_sh.at[dst_v.at[j]],
                              sem_s[b]).wait()

    pltpu.sync_copy(zeros_hbm, acc_sh.at[pl.ds(sid * rpt, rpt)])
    plsc.subcore_barrier()

    def seg_body(s, carry):
        pltpu.sync_copy(
            src_hbm.at[pl.ds((cid * NS + sid) * cpt + s * SEG, SEG)], src_v)
        pltpu.sync_copy(dst_hbm.at[pl.ds(sid * cpt + s * SEG, SEG)], dst_v)

        LAG = NBUF // 2
        for p in range(LAG):
            start_gather(p, p)

        def body(j0, c2):
            for u in range(NBUF):
                j = NBUF * j0 + u
                b = u

                @pl.when(j >= LAG)
                def _():
                    wait_scatter(j - LAG, (u - LAG) % NBUF)

                @pl.when(j < SEG - LAG)
                def _():
                    start_gather(j + LAG, (u + LAG) % NBUF)

                wait_gather(j, b)
                start_scatter(j, b)
            return c2

        lax.fori_loop(0, SEG // NBUF, body, 0)
        for p in range(LAG, 0, -1):
            wait_scatter(SEG - p, (SEG - p) % NBUF)
        return carry

    lax.fori_loop(0, cpt // SEG, seg_body, 0)
    plsc.subcore_barrier()
    pltpu.sync_copy(acc_sh.at[pl.ds(sid * rpt, rpt)],
                    out_hbm.at[cid, pl.ds(sid * rpt, rpt)])


def _tc_matmul_body(h, acc_ref, x_ref, w_ref, o_ref):
    a0 = acc_ref[0]
    a1 = acc_ref[1]
    g_lo = a0[:, :HALF]
    deg = a0[:, HALF:HALF + 1]
    g_hi = a1[:, :HALF]
    o_ref[...] = (
        jnp.dot(g_lo, w_ref[:HALF], preferred_element_type=jnp.float32)
        + jnp.dot(g_hi, w_ref[HALF:h], preferred_element_type=jnp.float32)
        + jnp.dot(deg * x_ref[...], w_ref[h:], preferred_element_type=jnp.float32)
    )


def kernel(x, edge_index, W):
    n, h = x.shape
    e = edge_index.shape[1]

    rpt = (-(-(n + 1) // NS) + 7) // 8 * 8
    n_acc = rpt * NS

    cpt = -(-(-(-(2 * e) // (NS * CHUNK))) // SEG) * SEG
    pad_e = NS * cpt * CHUNK - 2 * e
    src = edge_index[0].astype(jnp.int32)
    dst = edge_index[1].astype(jnp.int32)
    pad = jnp.full((pad_e,), n, jnp.int32)
    src_all = jnp.concatenate([src, dst, pad]).reshape(NS * cpt, CHUNK)
    dst_all = jnp.concatenate([dst, src, pad]).reshape(NS * cpt, CHUNK)
    src_both = jnp.concatenate([src_all, src_all + n_acc], axis=0)

    xcat = (jnp.zeros((2 * n_acc, WSC), jnp.float32)
            .at[:n, :HALF].set(x[:, :HALF])
            .at[:n, HALF].set(1.0)
            .at[n_acc:n_acc + n, :HALF].set(x[:, HALF:]))
    zeros_blk = jnp.zeros((rpt, WSC), jnp.float32)

    sc_scatter = pl.kernel(
        functools.partial(_sc_scatter_body, cpt, rpt),
        out_type=jax.ShapeDtypeStruct((NC, n_acc, WSC), jnp.float32),
        mesh=plsc.VectorSubcoreMesh(core_axis_name="c", subcore_axis_name="s",
                                    num_cores=NC, num_subcores=NS),
        scratch_types=[
            pltpu.VMEM((SEG, CHUNK), jnp.int32),
            pltpu.VMEM((SEG, CHUNK), jnp.int32),
            pltpu.VMEM((NBUF, CHUNK, WSC), jnp.float32),
            pltpu.VMEM_SHARED((n_acc, WSC), jnp.float32),
        ] + [pltpu.SemaphoreType.DMA] * (2 * NBUF),
        compiler_params=pltpu.CompilerParams(use_tc_tiling_on_sc=False),
    )
    acc = sc_scatter(xcat, src_both, dst_all, zeros_blk)

    x_pad = jnp.zeros((n_acc, h), jnp.float32).at[:n].set(x)
    out = pl.pallas_call(
        functools.partial(_tc_matmul_body, h),
        grid=(NS,),
        in_specs=[
            pl.BlockSpec((NC, rpt, WSC), lambda i: (0, i, 0)),
            pl.BlockSpec((rpt, h), lambda i: (i, 0)),
            pl.BlockSpec((2 * h, h), lambda i: (0, 0)),
        ],
        out_specs=pl.BlockSpec((rpt, h), lambda i: (i, 0)),
        out_shape=jax.ShapeDtypeStruct((n_acc, h), jnp.float32),
    )(acc, x_pad, W)
    return out[:n]


# speedup vs baseline: 2.3153x; 2.0428x over previous
import functools

import jax
import jax.numpy as jnp
from jax import lax
from jax.experimental import pallas as pl
from jax.experimental.pallas import tpu as pltpu
from jax.experimental.pallas import tpu_sc as plsc

NC = 2
NS = 16
CHUNK = 128
WSC = 80
HALF = 64


SEG = 8


NBUF = 2


def _sc_scatter_body(cpt, rpt, xcat_hbm, src_hbm, dst_hbm, zeros_hbm, out_hbm,
                     src_v, dst_v, rows_v, tbl_sh, acc_sh, *sems):
    cid = lax.axis_index("c")
    sid = lax.axis_index("s")
    sem_g = sems[:NBUF]
    sem_s = sems[NBUF:]

    def start_gather(j, b):
        pltpu.async_copy(tbl_sh.at[src_v.at[j]], rows_v.at[b], sem_g[b])

    def wait_gather(j, b):
        pltpu.make_async_copy(tbl_sh.at[src_v.at[j]], rows_v.at[b],
                              sem_g[b]).wait()

    def start_scatter(j, b):
        pltpu.async_copy(rows_v.at[b], acc_sh.at[dst_v.at[j]], sem_s[b],
                         add=True)

    def wait_scatter(j, b):
        pltpu.make_async_copy(rows_v.at[b], acc_sh.at[dst_v.at[j]],
                              sem_s[b]).wait()

    pltpu.sync_copy(xcat_hbm.at[pl.ds(cid * (rpt * NS) + sid * rpt, rpt)],
                    tbl_sh.at[pl.ds(sid * rpt, rpt)])
    pltpu.sync_copy(zeros_hbm, acc_sh.at[pl.ds(sid * rpt, rpt)])
    plsc.subcore_barrier()

    def seg_body(s, carry):
        pltpu.sync_copy(src_hbm.at[pl.ds(sid * cpt + s * SEG, SEG)], src_v)
        pltpu.sync_copy(dst_hbm.at[pl.ds(sid * cpt + s * SEG, SEG)], dst_v)

        LAG = max(NBUF // 2, 1)
        for p in range(LAG):
            start_gather(p, p)

        def body(j0, c2):
            for u in range(NBUF):
                j = NBUF * j0 + u
                b = u

                @pl.when(j >= LAG)
                def _():
                    wait_scatter(j - LAG, (u - LAG) % NBUF)

                @pl.when(j < SEG - LAG)
                def _():
                    start_gather(j + LAG, (u + LAG) % NBUF)

                wait_gather(j, b)
                start_scatter(j, b)
            return c2

        lax.fori_loop(0, SEG // NBUF, body, 0)
        for p in range(LAG, 0, -1):
            wait_scatter(SEG - p, (SEG - p) % NBUF)
        return carry

    lax.fori_loop(0, cpt // SEG, seg_body, 0)
    plsc.subcore_barrier()
    pltpu.sync_copy(acc_sh.at[pl.ds(sid * rpt, rpt)],
                    out_hbm.at[cid, pl.ds(sid * rpt, rpt)])


def _tc_matmul_body(h, acc_ref, x_ref, w_ref, o_ref):
    a0 = acc_ref[0]
    a1 = acc_ref[1]
    g_lo = a0[:, :HALF]
    deg = a0[:, HALF:HALF + 1]
    g_hi = a1[:, :HALF]
    o_ref[...] = (
        jnp.dot(g_lo, w_ref[:HALF], preferred_element_type=jnp.float32)
        + jnp.dot(g_hi, w_ref[HALF:h], preferred_element_type=jnp.float32)
        + jnp.dot(deg * x_ref[...], w_ref[h:], preferred_element_type=jnp.float32)
    )


def kernel(x, edge_index, W):
    n, h = x.shape
    e = edge_index.shape[1]

    rpt = (-(-(n + 1) // NS) + 7) // 8 * 8
    n_acc = rpt * NS

    cpt = -(-(-(-(2 * e) // (NS * CHUNK))) // SEG) * SEG
    pad_e = NS * cpt * CHUNK - 2 * e
    src = edge_index[0].astype(jnp.int32)
    dst = edge_index[1].astype(jnp.int32)
    pad = jnp.full((pad_e,), n, jnp.int32)
    src_all = jnp.concatenate([src, dst, pad]).reshape(NS * cpt, CHUNK)
    dst_all = jnp.concatenate([dst, src, pad]).reshape(NS * cpt, CHUNK)
    xcat = (jnp.zeros((2 * n_acc, WSC), jnp.float32)
            .at[:n, :HALF].set(x[:, :HALF])
            .at[:n, HALF].set(1.0)
            .at[n_acc:n_acc + n, :HALF].set(x[:, HALF:]))
    zeros_blk = jnp.zeros((rpt, WSC), jnp.float32)

    sc_scatter = pl.kernel(
        functools.partial(_sc_scatter_body, cpt, rpt),
        out_type=jax.ShapeDtypeStruct((NC, n_acc, WSC), jnp.float32),
        mesh=plsc.VectorSubcoreMesh(core_axis_name="c", subcore_axis_name="s",
                                    num_cores=NC, num_subcores=NS),
        scratch_types=[
            pltpu.VMEM((SEG, CHUNK), jnp.int32),
            pltpu.VMEM((SEG, CHUNK), jnp.int32),
            pltpu.VMEM((NBUF, CHUNK, WSC), jnp.float32),
            pltpu.VMEM_SHARED((n_acc, WSC), jnp.float32),
            pltpu.VMEM_SHARED((n_acc, WSC), jnp.float32),
        ] + [pltpu.SemaphoreType.DMA] * (2 * NBUF),
        compiler_params=pltpu.CompilerParams(use_tc_tiling_on_sc=False),
    )
    acc = sc_scatter(xcat, src_all, dst_all, zeros_blk)

    x_pad = jnp.zeros((n_acc, h), jnp.float32).at[:n].set(x)
    out = pl.pallas_call(
        functools.partial(_tc_matmul_body, h),
        grid=(NS,),
        in_specs=[
            pl.BlockSpec((NC, rpt, WSC), lambda i: (0, i, 0)),
            pl.BlockSpec((rpt, h), lambda i: (i, 0)),
            pl.BlockSpec((2 * h, h), lambda i: (0, 0)),
        ],
        out_specs=pl.BlockSpec((rpt, h), lambda i: (i, 0)),
        out_shape=jax.ShapeDtypeStruct((n_acc, h), jnp.float32),
    )(acc, x_pad, W)
    return out[:n]


# speedup vs baseline: 2.7012x; 1.1666x over previous
import functools

import jax
import jax.numpy as jnp
from jax import lax
from jax.experimental import pallas as pl
from jax.experimental.pallas import tpu as pltpu
from jax.experimental.pallas import tpu_sc as plsc

NC = 2
NS = 16
CHUNK = 128
WSC = 72
HALF = 64


SEG = 8


NBUF = 4


def _sc_scatter_body(cpt, rpt, xcat_hbm, src_hbm, dst_hbm, zeros_hbm, out_hbm,
                     src_v, dst_v, rows_v, tbl_sh, acc_sh, *sems):
    cid = lax.axis_index("c")
    sid = lax.axis_index("s")
    sem_g = sems[:NBUF]
    sem_s = sems[NBUF:]

    def start_gather(j, b):
        pltpu.async_copy(tbl_sh.at[src_v.at[j]], rows_v.at[b], sem_g[b])

    def wait_gather(j, b):
        pltpu.make_async_copy(tbl_sh.at[src_v.at[j]], rows_v.at[b],
                              sem_g[b]).wait()

    def start_scatter(j, b):
        pltpu.async_copy(rows_v.at[b], acc_sh.at[dst_v.at[j]], sem_s[b],
                         add=True)

    def wait_scatter(j, b):
        pltpu.make_async_copy(rows_v.at[b], acc_sh.at[dst_v.at[j]],
                              sem_s[b]).wait()

    pltpu.sync_copy(xcat_hbm.at[pl.ds(cid * (rpt * NS) + sid * rpt, rpt)],
                    tbl_sh.at[pl.ds(sid * rpt, rpt)])
    pltpu.sync_copy(zeros_hbm, acc_sh.at[pl.ds(sid * rpt, rpt)])
    plsc.subcore_barrier()

    def seg_body(s, carry):
        pltpu.sync_copy(src_hbm.at[pl.ds(sid * cpt + s * SEG, SEG)], src_v)
        pltpu.sync_copy(dst_hbm.at[pl.ds(sid * cpt + s * SEG, SEG)], dst_v)

        LAG = max(NBUF // 2, 1)
        for p in range(LAG):
            start_gather(p, p)

        def body(j0, c2):
            for u in range(NBUF):
                j = NBUF * j0 + u
                b = u

                @pl.when(j >= LAG)
                def _():
                    wait_scatter(j - LAG, (u - LAG) % NBUF)

                @pl.when(j < SEG - LAG)
                def _():
                    start_gather(j + LAG, (u + LAG) % NBUF)

                wait_gather(j, b)
                start_scatter(j, b)
            return c2

        lax.fori_loop(0, SEG // NBUF, body, 0)
        for p in range(LAG, 0, -1):
            wait_scatter(SEG - p, (SEG - p) % NBUF)
        return carry

    lax.fori_loop(0, cpt // SEG, seg_body, 0)
    plsc.subcore_barrier()
    pltpu.sync_copy(acc_sh.at[pl.ds(sid * rpt, rpt)],
                    out_hbm.at[cid, pl.ds(sid * rpt, rpt)])


def _tc_matmul_body(h, acc_ref, x_ref, w_ref, o_ref):
    a0 = acc_ref[0]
    a1 = acc_ref[1]
    g_lo = a0[:, :HALF]
    deg = a0[:, HALF:HALF + 1]
    g_hi = a1[:, :HALF]
    o_ref[...] = (
        jnp.dot(g_lo, w_ref[:HALF], preferred_element_type=jnp.float32)
        + jnp.dot(g_hi, w_ref[HALF:h], preferred_element_type=jnp.float32)
        + jnp.dot(deg * x_ref[...], w_ref[h:], preferred_element_type=jnp.float32)
    )


def kernel(x, edge_index, W):
    n, h = x.shape
    e = edge_index.shape[1]

    rpt = (-(-(n + 1) // NS) + 7) // 8 * 8
    n_acc = rpt * NS

    cpt = -(-(-(-(2 * e) // (NS * CHUNK))) // SEG) * SEG
    pad_e = NS * cpt * CHUNK - 2 * e
    src = edge_index[0].astype(jnp.int32)
    dst = edge_index[1].astype(jnp.int32)
    pad = jnp.full((pad_e,), n, jnp.int32)
    src_all = jnp.concatenate([src, dst, pad]).reshape(NS * cpt, CHUNK)
    dst_all = jnp.concatenate([dst, src, pad]).reshape(NS * cpt, CHUNK)
    xcat = (jnp.zeros((2 * n_acc, WSC), jnp.float32)
            .at[:n, :HALF].set(x[:, :HALF])
            .at[:n, HALF].set(1.0)
            .at[n_acc:n_acc + n, :HALF].set(x[:, HALF:]))
    zeros_blk = jnp.zeros((rpt, WSC), jnp.float32)

    sc_scatter = pl.kernel(
        functools.partial(_sc_scatter_body, cpt, rpt),
        out_type=jax.ShapeDtypeStruct((NC, n_acc, WSC), jnp.float32),
        mesh=plsc.VectorSubcoreMesh(core_axis_name="c", subcore_axis_name="s",
                                    num_cores=NC, num_subcores=NS),
        scratch_types=[
            pltpu.VMEM((SEG, CHUNK), jnp.int32),
            pltpu.VMEM((SEG, CHUNK), jnp.int32),
            pltpu.VMEM((NBUF, CHUNK, WSC), jnp.float32),
            pltpu.VMEM_SHARED((n_acc, WSC), jnp.float32),
            pltpu.VMEM_SHARED((n_acc, WSC), jnp.float32),
        ] + [pltpu.SemaphoreType.DMA] * (2 * NBUF),
        compiler_params=pltpu.CompilerParams(use_tc_tiling_on_sc=False),
    )
    acc = sc_scatter(xcat, src_all, dst_all, zeros_blk)

    x_pad = jnp.zeros((n_acc, h), jnp.float32).at[:n].set(x)
    out = pl.pallas_call(
        functools.partial(_tc_matmul_body, h),
        grid=(NS,),
        in_specs=[
            pl.BlockSpec((NC, rpt, WSC), lambda i: (0, i, 0)),
            pl.BlockSpec((rpt, h), lambda i: (i, 0)),
            pl.BlockSpec((2 * h, h), lambda i: (0, 0)),
        ],
        out_specs=pl.BlockSpec((rpt, h), lambda i: (i, 0)),
        out_shape=jax.ShapeDtypeStruct((n_acc, h), jnp.float32),
    )(acc, x_pad, W)
    return out[:n]


# speedup vs baseline: 3.2294x; 1.1955x over previous
import functools

import jax
import jax.numpy as jnp
from jax import lax
from jax.experimental import pallas as pl
from jax.experimental.pallas import tpu as pltpu
from jax.experimental.pallas import tpu_sc as plsc

NC = 2
NS = 16
CHUNK = 128
WSC = 72
HALF = 64
SEGC = 4
NBUF = 4


def _sc_scatter_body(n, ept, rpt, x_hbm, ei_hbm, zeros_hbm, ones_hbm, out_hbm,
                     idx_a, idx_b, tidx_a, tidx_b, rows_v, tbl_sh, acc_sh,
                     *sems):
    cid = lax.axis_index("c")
    sid = lax.axis_index("s")
    sem_g = sems[:NBUF]
    sem_s = sems[NBUF:]
    nfull = ept // CHUNK
    tail = ept - nfull * CHUNK
    nseg = nfull // SEGC
    idx = (idx_a, idx_b)

    def start_gather(j, p, b):
        pltpu.async_copy(tbl_sh.at[idx[p].at[pl.ds(j * CHUNK, CHUNK)]],
                         rows_v.at[b], sem_g[b])

    def wait_gather(j, p, b):
        pltpu.make_async_copy(tbl_sh.at[idx[p].at[pl.ds(j * CHUNK, CHUNK)]],
                              rows_v.at[b], sem_g[b]).wait()

    def start_scatter(j, p, b):
        pltpu.async_copy(rows_v.at[b],
                         acc_sh.at[idx[1 - p].at[pl.ds(j * CHUNK, CHUNK)]],
                         sem_s[b], add=True)

    def wait_scatter(j, p, b):
        pltpu.make_async_copy(rows_v.at[b],
                              acc_sh.at[idx[1 - p].at[pl.ds(j * CHUNK, CHUNK)]],
                              sem_s[b]).wait()

    row0 = sid * rpt
    short = n - (NS - 1) * rpt

    @pl.when(row0 + rpt <= n)
    def _():
        pltpu.sync_copy(x_hbm.at[pl.ds(row0, rpt), pl.ds(cid * HALF, HALF)],
                        tbl_sh.at[pl.ds(row0, rpt), pl.ds(0, HALF)])
        pltpu.sync_copy(ones_hbm, tbl_sh.at[pl.ds(row0, rpt), pl.ds(HALF, 8)])

    @pl.when(row0 + rpt > n)
    def _():
        pltpu.sync_copy(x_hbm.at[pl.ds(row0, short), pl.ds(cid * HALF, HALF)],
                        tbl_sh.at[pl.ds(row0, short), pl.ds(0, HALF)])
        pltpu.sync_copy(ones_hbm.at[pl.ds(0, short)],
                        tbl_sh.at[pl.ds(row0, short), pl.ds(HALF, 8)])

    pltpu.sync_copy(zeros_hbm, acc_sh.at[pl.ds(sid * rpt, rpt)])
    plsc.subcore_barrier()

    ebase = sid * ept

    def seg_body(s, carry):
        off = ebase + s * (SEGC * CHUNK)
        pltpu.sync_copy(ei_hbm.at[0, pl.ds(off, SEGC * CHUNK)], idx_a)
        pltpu.sync_copy(ei_hbm.at[1, pl.ds(off, SEGC * CHUNK)], idx_b)

        LAG = NBUF // 2
        NSTEP = 2 * SEGC
        for k in range(LAG):
            start_gather(k // 2, k % 2, k % NBUF)

        for k in range(NSTEP):
            b = k % NBUF
            if k >= LAG:
                wait_scatter((k - LAG) // 2, (k - LAG) % 2, (k - LAG) % NBUF)
            if k < NSTEP - LAG:
                start_gather((k + LAG) // 2, (k + LAG) % 2, (k + LAG) % NBUF)
            wait_gather(k // 2, k % 2, b)
            start_scatter(k // 2, k % 2, b)

        for k in range(NSTEP - LAG, NSTEP):
            wait_scatter(k // 2, k % 2, k % NBUF)
        return carry

    lax.fori_loop(0, nseg, seg_body, 0)

    if tail:
        toff = ebase + nfull * CHUNK
        pltpu.sync_copy(ei_hbm.at[0, pl.ds(toff, tail)], tidx_a)
        pltpu.sync_copy(ei_hbm.at[1, pl.ds(toff, tail)], tidx_b)
        tidx = (tidx_a, tidx_b)
        for p in range(2):
            pltpu.async_copy(tbl_sh.at[tidx[p]],
                             rows_v.at[p, pl.ds(0, tail)], sem_g[p])
        for p in range(2):
            pltpu.make_async_copy(tbl_sh.at[tidx[p]],
                                  rows_v.at[p, pl.ds(0, tail)],
                                  sem_g[p]).wait()
            pltpu.async_copy(rows_v.at[p, pl.ds(0, tail)],
                             acc_sh.at[tidx[1 - p]], sem_s[p], add=True)
        for p in range(2):
            pltpu.make_async_copy(rows_v.at[p, pl.ds(0, tail)],
                                  acc_sh.at[tidx[1 - p]], sem_s[p]).wait()

    plsc.subcore_barrier()
    pltpu.sync_copy(acc_sh.at[pl.ds(sid * rpt, rpt)],
                    out_hbm.at[cid, pl.ds(sid * rpt, rpt)])


def _tc_matmul_body(h, acc_ref, x_ref, w_ref, o_ref):
    a0 = acc_ref[0]
    a1 = acc_ref[1]
    g_lo = a0[:, :HALF]
    deg = a0[:, HALF:HALF + 1]
    g_hi = a1[:, :HALF]
    o_ref[...] = (
        jnp.dot(g_lo, w_ref[:HALF], preferred_element_type=jnp.float32)
        + jnp.dot(g_hi, w_ref[HALF:h], preferred_element_type=jnp.float32)
        + jnp.dot(deg * x_ref[...], w_ref[h:], preferred_element_type=jnp.float32)
    )


def kernel(x, edge_index, W):
    n, h = x.shape
    e = edge_index.shape[1]

    rpt = (-(-n // NS) + 7) // 8 * 8
    n_acc = rpt * NS
    ept = e // NS
    assert ept * NS == e and ept % 8 == 0
    tail = ept - (ept // CHUNK) * CHUNK

    ei = edge_index.astype(jnp.int32)
    zeros_blk = jnp.zeros((rpt, WSC), jnp.float32)
    ones_col = jnp.ones((rpt, 8), jnp.float32)

    sc_scatter = pl.kernel(
        functools.partial(_sc_scatter_body, n, ept, rpt),
        out_type=jax.ShapeDtypeStruct((NC, n_acc, WSC), jnp.float32),
        mesh=plsc.VectorSubcoreMesh(core_axis_name="c", subcore_axis_name="s",
                                    num_cores=NC, num_subcores=NS),
        scratch_types=[
            pltpu.VMEM((SEGC * CHUNK,), jnp.int32),
            pltpu.VMEM((SEGC * CHUNK,), jnp.int32),
            pltpu.VMEM((max(tail, 8),), jnp.int32),
            pltpu.VMEM((max(tail, 8),), jnp.int32),
            pltpu.VMEM((NBUF, CHUNK, WSC), jnp.float32),
            pltpu.VMEM_SHARED((n_acc, WSC), jnp.float32),
            pltpu.VMEM_SHARED((n_acc, WSC), jnp.float32),
        ] + [pltpu.SemaphoreType.DMA] * (2 * NBUF),
        compiler_params=pltpu.CompilerParams(use_tc_tiling_on_sc=False),
    )
    acc = sc_scatter(x, ei, zeros_blk, ones_col)

    out = pl.pallas_call(
        functools.partial(_tc_matmul_body, h),
        grid=(NS,),
        in_specs=[
            pl.BlockSpec((NC, rpt, WSC), lambda i: (0, i, 0)),
            pl.BlockSpec((rpt, h), lambda i: (i, 0)),
            pl.BlockSpec((2 * h, h), lambda i: (0, 0)),
        ],
        out_specs=pl.BlockSpec((rpt, h), lambda i: (i, 0)),
        out_shape=jax.ShapeDtypeStruct((n_acc, h), jnp.float32),
    )(acc, x, W)
    return out[:n]


# speedup vs baseline: 3.8204x; 1.1830x over previous
import functools

import jax
import jax.numpy as jnp
from jax import lax
from jax.experimental import pallas as pl
from jax.experimental.pallas import tpu as pltpu
from jax.experimental.pallas import tpu_sc as plsc

NC = 2
NS = 16
CHUNK = 64
WSC = 72
HALF = 64
SEGC = 8
NBUF = 8


def _sc_scatter_body(n, ept, rpt, x_hbm, ei_hbm, zeros_hbm, ones_hbm, out_hbm,
                     idx_a, idx_b, tidx_a, tidx_b, rows_v, tbl_sh, acc_sh,
                     *sems):
    cid = lax.axis_index("c")
    sid = lax.axis_index("s")
    sem_g = sems[:NBUF]
    sem_s = sems[NBUF:]
    nfull = ept // CHUNK
    tail = ept - nfull * CHUNK
    nseg = nfull // SEGC
    idx = (idx_a, idx_b)

    def start_gather(j, p, b):
        pltpu.async_copy(tbl_sh.at[idx[p].at[pl.ds(j * CHUNK, CHUNK)]],
                         rows_v.at[b], sem_g[b])

    def wait_gather(j, p, b):
        pltpu.make_async_copy(tbl_sh.at[idx[p].at[pl.ds(j * CHUNK, CHUNK)]],
                              rows_v.at[b], sem_g[b]).wait()

    def start_scatter(j, p, b):
        pltpu.async_copy(rows_v.at[b],
                         acc_sh.at[idx[1 - p].at[pl.ds(j * CHUNK, CHUNK)]],
                         sem_s[b], add=True)

    def wait_scatter(j, p, b):
        pltpu.make_async_copy(rows_v.at[b],
                              acc_sh.at[idx[1 - p].at[pl.ds(j * CHUNK, CHUNK)]],
                              sem_s[b]).wait()

    row0 = sid * rpt
    short = n - (NS - 1) * rpt

    @pl.when(row0 + rpt <= n)
    def _():
        pltpu.sync_copy(x_hbm.at[pl.ds(row0, rpt), pl.ds(cid * HALF, HALF)],
                        tbl_sh.at[pl.ds(row0, rpt), pl.ds(0, HALF)])
        pltpu.sync_copy(ones_hbm, tbl_sh.at[pl.ds(row0, rpt), pl.ds(HALF, 8)])

    @pl.when(row0 + rpt > n)
    def _():
        pltpu.sync_copy(x_hbm.at[pl.ds(row0, short), pl.ds(cid * HALF, HALF)],
                        tbl_sh.at[pl.ds(row0, short), pl.ds(0, HALF)])
        pltpu.sync_copy(ones_hbm.at[pl.ds(0, short)],
                        tbl_sh.at[pl.ds(row0, short), pl.ds(HALF, 8)])

    pltpu.sync_copy(zeros_hbm, acc_sh.at[pl.ds(sid * rpt, rpt)])
    plsc.subcore_barrier()

    ebase = sid * ept

    def seg_body(s, carry):
        off = ebase + s * (SEGC * CHUNK)
        pltpu.sync_copy(ei_hbm.at[0, pl.ds(off, SEGC * CHUNK)], idx_a)
        pltpu.sync_copy(ei_hbm.at[1, pl.ds(off, SEGC * CHUNK)], idx_b)

        LAG = NBUF // 2
        NSTEP = 2 * SEGC
        for k in range(LAG):
            start_gather(k // 2, k % 2, k % NBUF)

        for k in range(NSTEP):
            b = k % NBUF
            if k >= LAG:
                wait_scatter((k - LAG) // 2, (k - LAG) % 2, (k - LAG) % NBUF)
            if k < NSTEP - LAG:
                start_gather((k + LAG) // 2, (k + LAG) % 2, (k + LAG) % NBUF)
            wait_gather(k // 2, k % 2, b)
            start_scatter(k // 2, k % 2, b)

        for k in range(NSTEP - LAG, NSTEP):
            wait_scatter(k // 2, k % 2, k % NBUF)
        return carry

    lax.fori_loop(0, nseg, seg_body, 0)

    if tail:
        toff = ebase + nfull * CHUNK
        pltpu.sync_copy(ei_hbm.at[0, pl.ds(toff, tail)], tidx_a)
        pltpu.sync_copy(ei_hbm.at[1, pl.ds(toff, tail)], tidx_b)
        tidx = (tidx_a, tidx_b)
        for p in range(2):
            pltpu.async_copy(tbl_sh.at[tidx[p]],
                             rows_v.at[p, pl.ds(0, tail)], sem_g[p])
        for p in range(2):
            pltpu.make_async_copy(tbl_sh.at[tidx[p]],
                                  rows_v.at[p, pl.ds(0, tail)],
                                  sem_g[p]).wait()
            pltpu.async_copy(rows_v.at[p, pl.ds(0, tail)],
                             acc_sh.at[tidx[1 - p]], sem_s[p], add=True)
        for p in range(2):
            pltpu.make_async_copy(rows_v.at[p, pl.ds(0, tail)],
                                  acc_sh.at[tidx[1 - p]], sem_s[p]).wait()

    plsc.subcore_barrier()
    pltpu.sync_copy(acc_sh.at[pl.ds(sid * rpt, rpt)],
                    out_hbm.at[cid, pl.ds(sid * rpt, rpt)])


def _tc_matmul_body(h, acc_ref, x_ref, w_ref, o_ref):
    a0 = acc_ref[0]
    a1 = acc_ref[1]
    g_lo = a0[:, :HALF]
    deg = a0[:, HALF:HALF + 1]
    g_hi = a1[:, :HALF]
    o_ref[...] = (
        jnp.dot(g_lo, w_ref[:HALF], preferred_element_type=jnp.float32)
        + jnp.dot(g_hi, w_ref[HALF:h], preferred_element_type=jnp.float32)
        + jnp.dot(deg * x_ref[...], w_ref[h:], preferred_element_type=jnp.float32)
    )


def kernel(x, edge_index, W):
    n, h = x.shape
    e = edge_index.shape[1]

    rpt = (-(-n // NS) + 7) // 8 * 8
    n_acc = rpt * NS
    ept = e // NS
    assert ept * NS == e and ept % 8 == 0
    tail = ept - (ept // CHUNK) * CHUNK

    ei = edge_index.astype(jnp.int32)
    zeros_blk = jnp.zeros((rpt, WSC), jnp.float32)
    ones_col = jnp.ones((rpt, 8), jnp.float32)

    sc_scatter = pl.kernel(
        functools.partial(_sc_scatter_body, n, ept, rpt),
        out_type=jax.ShapeDtypeStruct((NC, n_acc, WSC), jnp.float32),
        mesh=plsc.VectorSubcoreMesh(core_axis_name="c", subcore_axis_name="s",
                                    num_cores=NC, num_subcores=NS),
        scratch_types=[
            pltpu.VMEM((SEGC * CHUNK,), jnp.int32),
            pltpu.VMEM((SEGC * CHUNK,), jnp.int32),
            pltpu.VMEM((max(tail, 8),), jnp.int32),
            pltpu.VMEM((max(tail, 8),), jnp.int32),
            pltpu.VMEM((NBUF, CHUNK, WSC), jnp.float32),
            pltpu.VMEM_SHARED((n_acc, WSC), jnp.float32),
            pltpu.VMEM_SHARED((n_acc, WSC), jnp.float32),
        ] + [pltpu.SemaphoreType.DMA] * (2 * NBUF),
        compiler_params=pltpu.CompilerParams(use_tc_tiling_on_sc=False),
    )
    acc = sc_scatter(x, ei, zeros_blk, ones_col)

    out = pl.pallas_call(
        functools.partial(_tc_matmul_body, h),
        grid=(NS,),
        in_specs=[
            pl.BlockSpec((NC, rpt, WSC), lambda i: (0, i, 0)),
            pl.BlockSpec((rpt, h), lambda i: (i, 0)),
            pl.BlockSpec((2 * h, h), lambda i: (0, 0)),
        ],
        out_specs=pl.BlockSpec((rpt, h), lambda i: (i, 0)),
        out_shape=jax.ShapeDtypeStruct((n_acc, h), jnp.float32),
    )(acc, x, W)
    return out[:n]


# speedup vs baseline: 4.5776x; 1.1982x over previous
import functools

import jax
import jax.numpy as jnp
from jax import lax
from jax.experimental import pallas as pl
from jax.experimental.pallas import tpu as pltpu
from jax.experimental.pallas import tpu_sc as plsc

NC = 2
NS = 16
CHUNK = 64
WSC = 72
HALF = 64
SEGC = 8
NBUF = 8


def _sc_scatter_body(n, ept, rpt, x_hbm, ei_hbm, zeros_hbm, ones_hbm, out_hbm,
                     idx_a, idx_b, tidx_a, tidx_b, rows_v, tbl_sh, acc_sh,
                     sem_st, *sems):
    cid = lax.axis_index("c")
    sid = lax.axis_index("s")
    sem_g = sems[:NBUF]
    sem_s = sems[NBUF:]
    nfull = ept // CHUNK
    tail = ept - nfull * CHUNK
    nseg = nfull // SEGC
    idx = (idx_a, idx_b)

    def start_gather(j, p, b, q):
        pltpu.async_copy(tbl_sh.at[idx[p].at[q, pl.ds(j * CHUNK, CHUNK)]],
                         rows_v.at[b], sem_g[b])

    def wait_gather(j, p, b, q):
        pltpu.make_async_copy(tbl_sh.at[idx[p].at[q, pl.ds(j * CHUNK, CHUNK)]],
                              rows_v.at[b], sem_g[b]).wait()

    def start_scatter(j, p, b, q):
        pltpu.async_copy(rows_v.at[b],
                         acc_sh.at[idx[1 - p].at[q, pl.ds(j * CHUNK, CHUNK)]],
                         sem_s[b], add=True)

    def wait_scatter(j, p, b, q=0):
        pltpu.make_async_copy(rows_v.at[b],
                              acc_sh.at[idx[1 - p].at[0, pl.ds(0, CHUNK)]],
                              sem_s[b]).wait()

    row0 = sid * rpt
    short = n - (NS - 1) * rpt

    @pl.when(row0 + rpt <= n)
    def _():
        pltpu.sync_copy(x_hbm.at[pl.ds(row0, rpt), pl.ds(cid * HALF, HALF)],
                        tbl_sh.at[pl.ds(row0, rpt), pl.ds(0, HALF)])
        pltpu.sync_copy(ones_hbm, tbl_sh.at[pl.ds(row0, rpt), pl.ds(HALF, 8)])

    @pl.when(row0 + rpt > n)
    def _():
        pltpu.sync_copy(x_hbm.at[pl.ds(row0, short), pl.ds(cid * HALF, HALF)],
                        tbl_sh.at[pl.ds(row0, short), pl.ds(0, HALF)])
        pltpu.sync_copy(ones_hbm.at[pl.ds(0, short)],
                        tbl_sh.at[pl.ds(row0, short), pl.ds(HALF, 8)])

    pltpu.sync_copy(zeros_hbm, acc_sh.at[pl.ds(sid * rpt, rpt)])
    plsc.subcore_barrier()

    ebase = sid * ept
    LAG = NBUF // 2
    NSTEP = 2 * SEGC
    SEGE = SEGC * CHUNK

    def stage(s, q):
        off = ebase + s * SEGE
        pltpu.async_copy(ei_hbm.at[0, pl.ds(off, SEGE)], idx_a.at[q], sem_st)
        pltpu.async_copy(ei_hbm.at[1, pl.ds(off, SEGE)], idx_b.at[q], sem_st)

    def wait_stage(q):
        pltpu.make_async_copy(ei_hbm.at[0, pl.ds(ebase, SEGE)], idx_a.at[q],
                              sem_st).wait()
        pltpu.make_async_copy(ei_hbm.at[1, pl.ds(ebase, SEGE)], idx_b.at[q],
                              sem_st).wait()

    stage(0, 0)
    wait_stage(0)
    for k in range(LAG):
        start_gather(k // 2, k % 2, k % NBUF, 0)

    def seg_body(s, carry):
        q = s % 2
        for k in range(NSTEP):
            b = k % NBUF
            if k >= LAG:
                wait_scatter(0, (k - LAG) % 2, (k - LAG) % NBUF)
            else:
                @pl.when(s > 0)
                def _():
                    wait_scatter(0, (k - LAG) % 2, (k - LAG) % NBUF)

            if k == LAG:
                @pl.when(s + 1 < nseg)
                def _():
                    stage(s + 1, 1 - q)

            kk = k + LAG
            if kk < NSTEP:
                start_gather(kk // 2, kk % 2, kk % NBUF, q)
            else:
                if kk == NSTEP:
                    @pl.when(s + 1 < nseg)
                    def _():
                        wait_stage(1 - q)

                @pl.when(s + 1 < nseg)
                def _():
                    start_gather((kk - NSTEP) // 2, kk % 2, kk % NBUF, 1 - q)

            wait_gather(k // 2, k % 2, b, q)
            start_scatter(k // 2, k % 2, b, q)
        return carry

    lax.fori_loop(0, nseg, seg_body, 0)
    for k in range(NSTEP - LAG, NSTEP):
        wait_scatter(0, k % 2, k % NBUF)

    if tail:
        toff = ebase + nfull * CHUNK
        pltpu.sync_copy(ei_hbm.at[0, pl.ds(toff, tail)], tidx_a)
        pltpu.sync_copy(ei_hbm.at[1, pl.ds(toff, tail)], tidx_b)
        tidx = (tidx_a, tidx_b)
        for p in range(2):
            pltpu.async_copy(tbl_sh.at[tidx[p]],
                             rows_v.at[p, pl.ds(0, tail)], sem_g[p])
        for p in range(2):
            pltpu.make_async_copy(tbl_sh.at[tidx[p]],
                                  rows_v.at[p, pl.ds(0, tail)],
                                  sem_g[p]).wait()
            pltpu.async_copy(rows_v.at[p, pl.ds(0, tail)],
                             acc_sh.at[tidx[1 - p]], sem_s[p], add=True)
        for p in range(2):
            pltpu.make_async_copy(rows_v.at[p, pl.ds(0, tail)],
                                  acc_sh.at[tidx[1 - p]], sem_s[p]).wait()

    plsc.subcore_barrier()
    pltpu.sync_copy(acc_sh.at[pl.ds(sid * rpt, rpt)],
                    out_hbm.at[cid, pl.ds(sid * rpt, rpt)])


def _tc_matmul_body(h, acc_ref, x_ref, w_ref, o_ref):
    a0 = acc_ref[0]
    a1 = acc_ref[1]
    g_lo = a0[:, :HALF]
    deg = a0[:, HALF:HALF + 1]
    g_hi = a1[:, :HALF]
    o_ref[...] = (
        jnp.dot(g_lo, w_ref[:HALF], preferred_element_type=jnp.float32)
        + jnp.dot(g_hi, w_ref[HALF:h], preferred_element_type=jnp.float32)
        + jnp.dot(deg * x_ref[...], w_ref[h:], preferred_element_type=jnp.float32)
    )


def kernel(x, edge_index, W):
    n, h = x.shape
    e = edge_index.shape[1]

    rpt = (-(-n // NS) + 7) // 8 * 8
    n_acc = rpt * NS
    ept = e // NS
    assert ept * NS == e and ept % 8 == 0
    tail = ept - (ept // CHUNK) * CHUNK

    ei = edge_index.astype(jnp.int32)
    zeros_blk = jnp.zeros((rpt, WSC), jnp.float32)
    ones_col = jnp.ones((rpt, 8), jnp.float32)

    sc_scatter = pl.kernel(
        functools.partial(_sc_scatter_body, n, ept, rpt),
        out_type=jax.ShapeDtypeStruct((NC, n_acc, WSC), jnp.float32),
        mesh=plsc.VectorSubcoreMesh(core_axis_name="c", subcore_axis_name="s",
                                    num_cores=NC, num_subcores=NS),
        scratch_types=[
            pltpu.VMEM((2, SEGC * CHUNK), jnp.int32),
            pltpu.VMEM((2, SEGC * CHUNK), jnp.int32),
            pltpu.VMEM((max(tail, 8),), jnp.int32),
            pltpu.VMEM((max(tail, 8),), jnp.int32),
            pltpu.VMEM((NBUF, CHUNK, WSC), jnp.float32),
            pltpu.VMEM_SHARED((n_acc, WSC), jnp.float32),
            pltpu.VMEM_SHARED((n_acc, WSC), jnp.float32),
            pltpu.SemaphoreType.DMA,
        ] + [pltpu.SemaphoreType.DMA] * (2 * NBUF),
        compiler_params=pltpu.CompilerParams(use_tc_tiling_on_sc=False),
    )
    acc = sc_scatter(x, ei, zeros_blk, ones_col)

    out = pl.pallas_call(
        functools.partial(_tc_matmul_body, h),
        grid=(NS,),
        in_specs=[
            pl.BlockSpec((NC, rpt, WSC), lambda i: (0, i, 0)),
            pl.BlockSpec((rpt, h), lambda i: (i, 0)),
            pl.BlockSpec((2 * h, h), lambda i: (0, 0)),
        ],
        out_specs=pl.BlockSpec((rpt, h), lambda i: (i, 0)),
        out_shape=jax.ShapeDtypeStruct((n_acc, h), jnp.float32),
    )(acc, x, W)
    return out[:n]


# speedup vs baseline: 4.6458x; 1.0149x over previous
import functools

import jax
import jax.numpy as jnp
from jax import lax
from jax.experimental import pallas as pl
from jax.experimental.pallas import tpu as pltpu
from jax.experimental.pallas import tpu_sc as plsc

NC = 2
NS = 16
CHUNK = 64
WSC = 72
HALF = 64
SEGC = 8
NBUF = 8


def _sc_scatter_body(n, ept, rpt, x_hbm, ei_hbm, zeros_hbm, ones_hbm, out_hbm,
                     idx_a, idx_b, tidx_a, tidx_b, rows_v, tbl_sh, acc_sh,
                     sem_st, *sems):
    cid = lax.axis_index("c")
    sid = lax.axis_index("s")
    sem_g = sems[:NBUF]
    sem_s = sems[NBUF:]
    nfull = ept // CHUNK
    tail = ept - nfull * CHUNK
    nseg = nfull // SEGC
    idx = (idx_a, idx_b)

    def start_gather(j, p, b, q):
        pltpu.async_copy(tbl_sh.at[idx[p].at[q, pl.ds(j * CHUNK, CHUNK)]],
                         rows_v.at[b], sem_g[b])

    def wait_gather(j, p, b, q):
        pltpu.make_async_copy(tbl_sh.at[idx[p].at[q, pl.ds(j * CHUNK, CHUNK)]],
                              rows_v.at[b], sem_g[b]).wait()

    def start_scatter(j, p, b, q):
        pltpu.async_copy(rows_v.at[b],
                         acc_sh.at[idx[1 - p].at[q, pl.ds(j * CHUNK, CHUNK)]],
                         sem_s[b], add=True)

    def wait_scatter(j, p, b, q=0):
        pltpu.make_async_copy(rows_v.at[b],
                              acc_sh.at[idx[1 - p].at[0, pl.ds(0, CHUNK)]],
                              sem_s[b]).wait()

    row0 = sid * rpt
    short = n - (NS - 1) * rpt

    @pl.when(row0 + rpt <= n)
    def _():
        pltpu.sync_copy(x_hbm.at[pl.ds(row0, rpt), pl.ds(cid * HALF, HALF)],
                        tbl_sh.at[pl.ds(row0, rpt), pl.ds(0, HALF)])
        pltpu.sync_copy(ones_hbm, tbl_sh.at[pl.ds(row0, rpt), pl.ds(HALF, 8)])

    @pl.when(row0 + rpt > n)
    def _():
        pltpu.sync_copy(x_hbm.at[pl.ds(row0, short), pl.ds(cid * HALF, HALF)],
                        tbl_sh.at[pl.ds(row0, short), pl.ds(0, HALF)])
        pltpu.sync_copy(ones_hbm.at[pl.ds(0, short)],
                        tbl_sh.at[pl.ds(row0, short), pl.ds(HALF, 8)])

    pltpu.sync_copy(zeros_hbm, acc_sh.at[pl.ds(sid * rpt, rpt)])
    plsc.subcore_barrier()

    ebase = sid * ept
    LAG = NBUF // 2
    NSTEP = 2 * SEGC
    SEGE = SEGC * CHUNK

    def stage(s, q):
        off = ebase + s * SEGE
        pltpu.async_copy(ei_hbm.at[0, pl.ds(off, SEGE)], idx_a.at[q], sem_st)
        pltpu.async_copy(ei_hbm.at[1, pl.ds(off, SEGE)], idx_b.at[q], sem_st)

    def wait_stage(q):
        pltpu.make_async_copy(ei_hbm.at[0, pl.ds(ebase, SEGE)], idx_a.at[q],
                              sem_st).wait()
        pltpu.make_async_copy(ei_hbm.at[1, pl.ds(ebase, SEGE)], idx_b.at[q],
                              sem_st).wait()

    stage(0, 0)
    wait_stage(0)
    for k in range(LAG):
        start_gather(k // 2, k % 2, k % NBUF, 0)

    def seg_body(s, carry):
        q = s % 2
        for k in range(NSTEP):
            b = k % NBUF
            if k >= LAG:
                wait_scatter(0, (k - LAG) % 2, (k - LAG) % NBUF)
            else:
                @pl.when(s > 0)
                def _():
                    wait_scatter(0, (k - LAG) % 2, (k - LAG) % NBUF)

            if k == LAG:
                @pl.when(s + 1 < nseg)
                def _():
                    stage(s + 1, 1 - q)

            kk = k + LAG
            if kk < NSTEP:
                start_gather(kk // 2, kk % 2, kk % NBUF, q)
            else:
                if kk == NSTEP:
                    @pl.when(s + 1 < nseg)
                    def _():
                        wait_stage(1 - q)

                @pl.when(s + 1 < nseg)
                def _():
                    start_gather((kk - NSTEP) // 2, kk % 2, kk % NBUF, 1 - q)

            wait_gather(k // 2, k % 2, b, q)
            start_scatter(k // 2, k % 2, b, q)
        return carry

    lax.fori_loop(0, nseg, seg_body, 0)
    for k in range(NSTEP - LAG, NSTEP):
        wait_scatter(0, k % 2, k % NBUF)

    if tail:
        toff = ebase + nfull * CHUNK
        pltpu.sync_copy(ei_hbm.at[0, pl.ds(toff, tail)], tidx_a)
        pltpu.sync_copy(ei_hbm.at[1, pl.ds(toff, tail)], tidx_b)
        tidx = (tidx_a, tidx_b)
        for p in range(2):
            pltpu.async_copy(tbl_sh.at[tidx[p]],
                             rows_v.at[p, pl.ds(0, tail)], sem_g[p])
        for p in range(2):
            pltpu.make_async_copy(tbl_sh.at[tidx[p]],
                                  rows_v.at[p, pl.ds(0, tail)],
                                  sem_g[p]).wait()
            pltpu.async_copy(rows_v.at[p, pl.ds(0, tail)],
                             acc_sh.at[tidx[1 - p]], sem_s[p], add=True)
        for p in range(2):
            pltpu.make_async_copy(rows_v.at[p, pl.ds(0, tail)],
                                  acc_sh.at[tidx[1 - p]], sem_s[p]).wait()

    plsc.subcore_barrier()
    pltpu.sync_copy(acc_sh.at[pl.ds(sid * rpt, rpt)],
                    out_hbm.at[cid, pl.ds(sid * rpt, rpt)])


def _tc_matmul_body(h, acc_ref, x_ref, w_ref, o_ref):
    a0 = acc_ref[0]
    a1 = acc_ref[1]
    g_lo = a0[:, :HALF]
    deg = a0[:, HALF:HALF + 1]
    g_hi = a1[:, :HALF]
    o_ref[...] = (
        jnp.dot(g_lo, w_ref[:HALF], preferred_element_type=jnp.float32)
        + jnp.dot(g_hi, w_ref[HALF:h], preferred_element_type=jnp.float32)
        + jnp.dot(deg * x_ref[...], w_ref[h:], preferred_element_type=jnp.float32)
    )


def kernel(x, edge_index, W):
    n, h = x.shape
    e = edge_index.shape[1]

    rpt = (-(-n // NS) + 7) // 8 * 8
    n_acc = rpt * NS
    ept = e // NS
    assert ept * NS == e and ept % 8 == 0
    tail = ept - (ept // CHUNK) * CHUNK

    ei = edge_index.astype(jnp.int32)
    zeros_blk = jnp.zeros((rpt, WSC), jnp.float32)
    ones_col = jnp.ones((rpt, 8), jnp.float32)

    sc_scatter = pl.kernel(
        functools.partial(_sc_scatter_body, n, ept, rpt),
        out_type=jax.ShapeDtypeStruct((NC, n_acc, WSC), jnp.float32),
        mesh=plsc.VectorSubcoreMesh(core_axis_name="c", subcore_axis_name="s",
                                    num_cores=NC, num_subcores=NS),
        scratch_types=[
            pltpu.VMEM((2, SEGC * CHUNK), jnp.int32),
            pltpu.VMEM((2, SEGC * CHUNK), jnp.int32),
            pltpu.VMEM((max(tail, 8),), jnp.int32),
            pltpu.VMEM((max(tail, 8),), jnp.int32),
            pltpu.VMEM((NBUF, CHUNK, WSC), jnp.float32),
            pltpu.VMEM_SHARED((n_acc, WSC), jnp.float32),
            pltpu.VMEM_SHARED((n_acc, WSC), jnp.float32),
            pltpu.SemaphoreType.DMA,
        ] + [pltpu.SemaphoreType.DMA] * (2 * NBUF),
        compiler_params=pltpu.CompilerParams(use_tc_tiling_on_sc=False),
    )
    acc = sc_scatter(x, ei, zeros_blk, ones_col)

    out = pl.pallas_call(
        functools.partial(_tc_matmul_body, h),
        grid=(NS,),
        in_specs=[
            pl.BlockSpec((NC, rpt, WSC), lambda i: (0, i, 0)),
            pl.BlockSpec((rpt, h), lambda i: (i, 0)),
            pl.BlockSpec((2 * h, h), lambda i: (0, 0)),
        ],
        out_specs=pl.BlockSpec((rpt, h), lambda i: (i, 0)),
        out_shape=jax.ShapeDtypeStruct((n, h), jnp.float32),
    )(acc, x, W)
    return out


# speedup vs baseline: 4.6507x; 1.0011x over previous
import functools

import jax
import jax.numpy as jnp
from jax import lax
from jax.experimental import pallas as pl
from jax.experimental.pallas import tpu as pltpu
from jax.experimental.pallas import tpu_sc as plsc

NC = 2
NS = 16
CHUNK = 64
WSC = 72
HALF = 64
SEGC = 8
NBUF = 8
LAG = NBUF // 2
NSTEP = 2 * SEGC
SEGE = SEGC * CHUNK


def _sc_scatter_body(n, ept, rpt, x_hbm, ei_hbm, zeros_hbm, ones_hbm, out_hbm,
                     idx_a, idx_b, tidx_a, tidx_b, rows_v, tbl_sh, acc_sh,
                     sem_st, *sems):
    cid = lax.axis_index("c")
    sid = lax.axis_index("s")
    sem_g = sems[:NBUF]
    sem_s = sems[NBUF:]
    nfull = ept // CHUNK
    tail = ept - nfull * CHUNK
    nseg = nfull // SEGC
    idx = (idx_a, idx_b)

    def start_gather(j, p, b, q):
        pltpu.async_copy(tbl_sh.at[idx[p].at[q, pl.ds(j * CHUNK, CHUNK)]],
                         rows_v.at[b], sem_g[b])

    def wait_gather(j, p, b, q):
        pltpu.make_async_copy(tbl_sh.at[idx[p].at[q, pl.ds(j * CHUNK, CHUNK)]],
                              rows_v.at[b], sem_g[b]).wait()

    def start_scatter(j, p, b, q):
        pltpu.async_copy(rows_v.at[b],
                         acc_sh.at[idx[1 - p].at[q, pl.ds(j * CHUNK, CHUNK)]],
                         sem_s[b], add=True)

    def wait_scatter(b):
        pltpu.make_async_copy(rows_v.at[b],
                              acc_sh.at[idx_a.at[0, pl.ds(0, CHUNK)]],
                              sem_s[b]).wait()

    row0 = sid * rpt
    short = n - (NS - 1) * rpt

    @pl.when(row0 + rpt <= n)
    def _():
        pltpu.sync_copy(x_hbm.at[pl.ds(row0, rpt), pl.ds(cid * HALF, HALF)],
                        tbl_sh.at[pl.ds(row0, rpt), pl.ds(0, HALF)])
        pltpu.sync_copy(ones_hbm, tbl_sh.at[pl.ds(row0, rpt), pl.ds(HALF, 8)])
        pltpu.sync_copy(zeros_hbm, acc_sh.at[pl.ds(row0, rpt)])

    @pl.when(row0 + rpt > n)
    def _():
        pltpu.sync_copy(x_hbm.at[pl.ds(row0, short), pl.ds(cid * HALF, HALF)],
                        tbl_sh.at[pl.ds(row0, short), pl.ds(0, HALF)])
        pltpu.sync_copy(ones_hbm.at[pl.ds(0, short)],
                        tbl_sh.at[pl.ds(row0, short), pl.ds(HALF, 8)])
        pltpu.sync_copy(zeros_hbm.at[pl.ds(0, short)],
                        acc_sh.at[pl.ds(row0, short)])

    ebase = sid * ept

    def stage(s, q):
        off = ebase + s * SEGE
        pltpu.async_copy(ei_hbm.at[0, pl.ds(off, SEGE)], idx_a.at[q], sem_st)
        pltpu.async_copy(ei_hbm.at[1, pl.ds(off, SEGE)], idx_b.at[q], sem_st)

    def wait_stage():
        pltpu.make_async_copy(ei_hbm.at[0, pl.ds(0, SEGE)], idx_a.at[0],
                              sem_st).wait()
        pltpu.make_async_copy(ei_hbm.at[1, pl.ds(0, SEGE)], idx_b.at[0],
                              sem_st).wait()

    stage(0, 0)
    wait_stage()
    plsc.subcore_barrier()
    for k in range(LAG):
        start_gather(k // 2, k % 2, k % NBUF, 0)

    def seg_body(s, carry):
        q = s % 2
        qn = (s + 1) % 2
        for k in range(NSTEP):
            b = k % NBUF
            if k >= LAG:
                wait_scatter((k - LAG) % NBUF)
            else:
                @pl.when(s > 0)
                def _():
                    wait_scatter((k - LAG) % NBUF)

            if k == LAG:
                @pl.when(s + 1 < nseg)
                def _():
                    stage(s + 1, qn)

            kk = k + LAG
            if kk < NSTEP:
                start_gather(kk // 2, kk % 2, kk % NBUF, q)
            else:
                if kk == NSTEP:
                    @pl.when(s + 1 < nseg)
                    def _():
                        wait_stage()

                @pl.when(s + 1 < nseg)
                def _():
                    start_gather((kk - NSTEP) // 2, kk % 2, kk % NBUF, qn)

            wait_gather(k // 2, k % 2, b, q)
            start_scatter(k // 2, k % 2, b, q)
        return carry

    lax.fori_loop(0, nseg, seg_body, 0)
    for k in range(NSTEP - LAG, NSTEP):
        wait_scatter(k % NBUF)

    if tail:
        toff = ebase + nfull * CHUNK
        pltpu.sync_copy(ei_hbm.at[0, pl.ds(toff, tail)], tidx_a)
        pltpu.sync_copy(ei_hbm.at[1, pl.ds(toff, tail)], tidx_b)
        tidx = (tidx_a, tidx_b)
        for p in range(2):
            pltpu.async_copy(tbl_sh.at[tidx[p]],
                             rows_v.at[p, pl.ds(0, tail)], sem_g[p])
        for p in range(2):
            pltpu.make_async_copy(tbl_sh.at[tidx[p]],
                                  rows_v.at[p, pl.ds(0, tail)],
                                  sem_g[p]).wait()
            pltpu.async_copy(rows_v.at[p, pl.ds(0, tail)],
                             acc_sh.at[tidx[1 - p]], sem_s[p], add=True)
        for p in range(2):
            pltpu.make_async_copy(rows_v.at[p, pl.ds(0, tail)],
                                  acc_sh.at[tidx[1 - p]], sem_s[p]).wait()

    plsc.subcore_barrier()
    @pl.when(row0 + rpt <= n)
    def _():
        pltpu.sync_copy(acc_sh.at[pl.ds(row0, rpt)],
                        out_hbm.at[cid, pl.ds(row0, rpt)])

    @pl.when(row0 + rpt > n)
    def _():
        pltpu.sync_copy(acc_sh.at[pl.ds(row0, short)],
                        out_hbm.at[cid, pl.ds(row0, short)])


def _tc_matmul_body(h, acc_ref, x_ref, w_ref, o_ref):
    a0 = acc_ref[0]
    a1 = acc_ref[1]
    g_lo = a0[:, :HALF]
    deg = a0[:, HALF:HALF + 1]
    g_hi = a1[:, :HALF]
    o_ref[...] = (
        jnp.dot(g_lo, w_ref[:HALF], preferred_element_type=jnp.float32)
        + jnp.dot(g_hi, w_ref[HALF:h], preferred_element_type=jnp.float32)
        + jnp.dot(deg * x_ref[...], w_ref[h:], preferred_element_type=jnp.float32)
    )


def kernel(x, edge_index, W):
    n, h = x.shape
    e = edge_index.shape[1]

    rpt = (-(-n // NS) + 7) // 8 * 8
    ept = e // NS
    assert ept * NS == e and ept % 16 == 0
    tail = ept - (ept // CHUNK) * CHUNK

    ei = edge_index.astype(jnp.int32)
    zeros_blk = jnp.zeros((rpt, WSC), jnp.float32)
    ones_col = jnp.ones((rpt, 8), jnp.float32)

    sc_scatter = pl.kernel(
        functools.partial(_sc_scatter_body, n, ept, rpt),
        out_type=jax.ShapeDtypeStruct((NC, n, WSC), jnp.float32),
        mesh=plsc.VectorSubcoreMesh(core_axis_name="c", subcore_axis_name="s",
                                    num_cores=NC, num_subcores=NS),
        scratch_types=[
            pltpu.VMEM((2, SEGE), jnp.int32),
            pltpu.VMEM((2, SEGE), jnp.int32),
            pltpu.VMEM((max(tail, 8),), jnp.int32),
            pltpu.VMEM((max(tail, 8),), jnp.int32),
            pltpu.VMEM((NBUF, CHUNK, WSC), jnp.float32),
            pltpu.VMEM_SHARED((n, WSC), jnp.float32),
            pltpu.VMEM_SHARED((n, WSC), jnp.float32),
            pltpu.SemaphoreType.DMA,
        ] + [pltpu.SemaphoreType.DMA] * (2 * NBUF),
        compiler_params=pltpu.CompilerParams(use_tc_tiling_on_sc=False),
    )
    acc = sc_scatter(x, ei, zeros_blk, ones_col)

    out = pl.pallas_call(
        functools.partial(_tc_matmul_body, h),
        grid=(NS,),
        in_specs=[
            pl.BlockSpec((NC, rpt, WSC), lambda i: (0, i, 0)),
            pl.BlockSpec((rpt, h), lambda i: (i, 0)),
            pl.BlockSpec((2 * h, h), lambda i: (0, 0)),
        ],
        out_specs=pl.BlockSpec((rpt, h), lambda i: (i, 0)),
        out_shape=jax.ShapeDtypeStruct((n, h), jnp.float32),
    )(acc, x, W)
    return out


# speedup vs baseline: 4.6509x; 1.0000x over previous
import functools

import jax
import jax.numpy as jnp
from jax import lax
from jax.experimental import pallas as pl
from jax.experimental.pallas import tpu as pltpu
from jax.experimental.pallas import tpu_sc as plsc

NC = 2
NS = 16
CHUNK = 64
WSC = 72
HALF = 64
SEGC = 8
NBUF = 8
LAG = NBUF // 2
NSTEP = 2 * SEGC
SEGE = SEGC * CHUNK


def _sc_scatter_body(n, ept, rpt, x_hbm, ei_hbm, zeros_hbm, ones_hbm, out_hbm,
                     idx_a, idx_b, tidx_a, tidx_b, rows_v, tbl_sh, acc_sh,
                     sem_st, *sems):
    cid = lax.axis_index("c")
    sid = lax.axis_index("s")
    sem_g = sems[:NBUF]
    sem_s = sems[NBUF:]
    nfull = (ept // CHUNK // SEGC) * SEGC
    tail = ept - nfull * CHUNK
    nseg = nfull // SEGC
    idx = (idx_a, idx_b)

    def start_gather(j, p, b, q):
        pltpu.async_copy(tbl_sh.at[idx[p].at[q, pl.ds(j * CHUNK, CHUNK)]],
                         rows_v.at[b], sem_g[b])

    def wait_gather(j, p, b, q):
        pltpu.make_async_copy(tbl_sh.at[idx[p].at[q, pl.ds(j * CHUNK, CHUNK)]],
                              rows_v.at[b], sem_g[b]).wait()

    def start_scatter(j, p, b, q):
        pltpu.async_copy(rows_v.at[b],
                         acc_sh.at[idx[1 - p].at[q, pl.ds(j * CHUNK, CHUNK)]],
                         sem_s[b], add=True)

    def wait_scatter(b):
        pltpu.make_async_copy(rows_v.at[b],
                              acc_sh.at[idx_a.at[0, pl.ds(0, CHUNK)]],
                              sem_s[b]).wait()

    row0 = sid * rpt
    short = n - (NS - 1) * rpt

    @pl.when(row0 + rpt <= n)
    def _():
        pltpu.sync_copy(x_hbm.at[pl.ds(row0, rpt), pl.ds(cid * HALF, HALF)],
                        tbl_sh.at[pl.ds(row0, rpt), pl.ds(0, HALF)])
        pltpu.sync_copy(ones_hbm, tbl_sh.at[pl.ds(row0, rpt), pl.ds(HALF, 8)])
        pltpu.sync_copy(zeros_hbm, acc_sh.at[pl.ds(row0, rpt)])

    @pl.when(row0 + rpt > n)
    def _():
        pltpu.sync_copy(x_hbm.at[pl.ds(row0, short), pl.ds(cid * HALF, HALF)],
                        tbl_sh.at[pl.ds(row0, short), pl.ds(0, HALF)])
        pltpu.sync_copy(ones_hbm.at[pl.ds(0, short)],
                        tbl_sh.at[pl.ds(row0, short), pl.ds(HALF, 8)])
        pltpu.sync_copy(zeros_hbm.at[pl.ds(0, short)],
                        acc_sh.at[pl.ds(row0, short)])

    ebase = sid * ept

    def stage(s, q):
        off = ebase + s * SEGE
        pltpu.async_copy(ei_hbm.at[0, pl.ds(off, SEGE)], idx_a.at[q], sem_st)
        pltpu.async_copy(ei_hbm.at[1, pl.ds(off, SEGE)], idx_b.at[q], sem_st)

    def wait_stage():
        pltpu.make_async_copy(ei_hbm.at[0, pl.ds(0, SEGE)], idx_a.at[0],
                              sem_st).wait()
        pltpu.make_async_copy(ei_hbm.at[1, pl.ds(0, SEGE)], idx_b.at[0],
                              sem_st).wait()

    stage(0, 0)
    wait_stage()
    plsc.subcore_barrier()
    for k in range(LAG):
        start_gather(k // 2, k % 2, k % NBUF, 0)

    def seg_body(s, carry):
        q = s % 2
        qn = (s + 1) % 2
        for k in range(NSTEP):
            b = k % NBUF
            if k >= LAG:
                wait_scatter((k - LAG) % NBUF)
            else:
                @pl.when(s > 0)
                def _():
                    wait_scatter((k - LAG) % NBUF)

            if k == LAG:
                @pl.when(s + 1 < nseg)
                def _():
                    stage(s + 1, qn)

            kk = k + LAG
            if kk < NSTEP:
                start_gather(kk // 2, kk % 2, kk % NBUF, q)
            else:
                if kk == NSTEP:
                    @pl.when(s + 1 < nseg)
                    def _():
                        wait_stage()

                @pl.when(s + 1 < nseg)
                def _():
                    start_gather((kk - NSTEP) // 2, kk % 2, kk % NBUF, qn)

            wait_gather(k // 2, k % 2, b, q)
            start_scatter(k // 2, k % 2, b, q)
        return carry

    lax.fori_loop(0, nseg, seg_body, 0)
    for k in range(NSTEP - LAG, NSTEP):
        wait_scatter(k % NBUF)

    if tail:
        toff = ebase + nfull * CHUNK
        pltpu.sync_copy(ei_hbm.at[0, pl.ds(toff, tail)], tidx_a)
        pltpu.sync_copy(ei_hbm.at[1, pl.ds(toff, tail)], tidx_b)
        tidx = (tidx_a, tidx_b)
        for p in range(2):
            pltpu.async_copy(tbl_sh.at[tidx[p]],
                             rows_v.at[p, pl.ds(0, tail)], sem_g[p])
        for p in range(2):
            pltpu.make_async_copy(tbl_sh.at[tidx[p]],
                                  rows_v.at[p, pl.ds(0, tail)],
                                  sem_g[p]).wait()
            pltpu.async_copy(rows_v.at[p, pl.ds(0, tail)],
                             acc_sh.at[tidx[1 - p]], sem_s[p], add=True)
        for p in range(2):
            pltpu.make_async_copy(rows_v.at[p, pl.ds(0, tail)],
                                  acc_sh.at[tidx[1 - p]], sem_s[p]).wait()

    plsc.subcore_barrier()
    @pl.when(row0 + rpt <= n)
    def _():
        pltpu.sync_copy(acc_sh.at[pl.ds(row0, rpt)],
                        out_hbm.at[cid, pl.ds(row0, rpt)])

    @pl.when(row0 + rpt > n)
    def _():
        pltpu.sync_copy(acc_sh.at[pl.ds(row0, short)],
                        out_hbm.at[cid, pl.ds(row0, short)])


def _tc_matmul_body(h, acc_ref, x_ref, w_ref, o_ref):
    a0 = acc_ref[0]
    a1 = acc_ref[1]
    g_lo = a0[:, :HALF]
    deg = a0[:, HALF:HALF + 1]
    g_hi = a1[:, :HALF]
    o_ref[...] = (
        jnp.dot(g_lo, w_ref[:HALF], preferred_element_type=jnp.float32)
        + jnp.dot(g_hi, w_ref[HALF:h], preferred_element_type=jnp.float32)
        + jnp.dot(deg * x_ref[...], w_ref[h:], preferred_element_type=jnp.float32)
    )


def kernel(x, edge_index, W):
    n, h = x.shape
    e = edge_index.shape[1]

    rpt = (-(-n // NS) + 7) // 8 * 8
    ept = e // NS
    assert ept * NS == e and ept % 16 == 0
    tail = ept - (ept // CHUNK // SEGC) * SEGC * CHUNK
    assert 0 < tail <= 128 and tail % 8 == 0

    ei = edge_index.astype(jnp.int32)
    zeros_blk = jnp.zeros((rpt, WSC), jnp.float32)
    ones_col = jnp.ones((rpt, 8), jnp.float32)

    sc_scatter = pl.kernel(
        functools.partial(_sc_scatter_body, n, ept, rpt),
        out_type=jax.ShapeDtypeStruct((NC, n, WSC), jnp.float32),
        mesh=plsc.VectorSubcoreMesh(core_axis_name="c", subcore_axis_name="s",
                                    num_cores=NC, num_subcores=NS),
        scratch_types=[
            pltpu.VMEM((2, SEGE), jnp.int32),
            pltpu.VMEM((2, SEGE), jnp.int32),
            pltpu.VMEM((max(tail, 8),), jnp.int32),
            pltpu.VMEM((max(tail, 8),), jnp.int32),
            pltpu.VMEM((NBUF, CHUNK, WSC), jnp.float32),
            pltpu.VMEM_SHARED((n, WSC), jnp.float32),
            pltpu.VMEM_SHARED((n, WSC), jnp.float32),
            pltpu.SemaphoreType.DMA,
        ] + [pltpu.SemaphoreType.DMA] * (2 * NBUF),
        compiler_params=pltpu.CompilerParams(use_tc_tiling_on_sc=False),
    )
    acc = sc_scatter(x, ei, zeros_blk, ones_col)

    out = pl.pallas_call(
        functools.partial(_tc_matmul_body, h),
        grid=(NS,),
        in_specs=[
            pl.BlockSpec((NC, rpt, WSC), lambda i: (0, i, 0)),
            pl.BlockSpec((rpt, h), lambda i: (i, 0)),
            pl.BlockSpec((2 * h, h), lambda i: (0, 0)),
        ],
        out_specs=pl.BlockSpec((rpt, h), lambda i: (i, 0)),
        out_shape=jax.ShapeDtypeStruct((n, h), jnp.float32),
    )(acc, x, W)
    return out


# speedup vs baseline: 4.7596x; 1.0234x over previous
import functools

import jax
import jax.numpy as jnp
from jax import lax
from jax.experimental import pallas as pl
from jax.experimental.pallas import tpu as pltpu
from jax.experimental.pallas import tpu_sc as plsc

NC = 2
NS = 16
CHUNK = 64
WSC = 72
HALF = 64
SEGC = 8
NBUF = 8
LAG = NBUF // 2
NSTEP = 2 * SEGC
SEGE = SEGC * CHUNK


def _sc_scatter_body(n, ept, rpt, x_hbm, ei_hbm, zeros_hbm, ones_hbm, out_hbm,
                     idx_a, idx_b, tidx_a, tidx_b, rows_v, tbl_sh, acc_sh,
                     sem_st, *sems):
    cid = lax.axis_index("c")
    sid = lax.axis_index("s")
    sem_g = sems[:NBUF]
    sem_s = sems[NBUF:]
    nfull = (ept // CHUNK // SEGC) * SEGC
    tail = ept - nfull * CHUNK
    nseg = nfull // SEGC
    idx = (idx_a, idx_b)

    def start_gather(j, p, b, q):
        pltpu.async_copy(tbl_sh.at[idx[p].at[q, pl.ds(j * CHUNK, CHUNK)]],
                         rows_v.at[b], sem_g[b])

    def wait_gather(j, p, b, q):
        pltpu.make_async_copy(tbl_sh.at[idx[p].at[q, pl.ds(j * CHUNK, CHUNK)]],
                              rows_v.at[b], sem_g[b]).wait()

    def start_scatter(j, p, b, q):
        pltpu.async_copy(rows_v.at[b],
                         acc_sh.at[idx[1 - p].at[q, pl.ds(j * CHUNK, CHUNK)]],
                         sem_s[b], add=True)

    def wait_scatter(b):
        pltpu.make_async_copy(rows_v.at[b],
                              acc_sh.at[idx_a.at[0, pl.ds(0, CHUNK)]],
                              sem_s[b]).wait()

    row0 = sid * rpt
    short = n - (NS - 1) * rpt

    @pl.when(row0 + rpt <= n)
    def _():
        pltpu.sync_copy(x_hbm.at[pl.ds(row0, rpt), pl.ds(cid * HALF, HALF)],
                        tbl_sh.at[pl.ds(row0, rpt), pl.ds(0, HALF)])
        pltpu.sync_copy(ones_hbm, tbl_sh.at[pl.ds(row0, rpt), pl.ds(HALF, 8)])
        pltpu.sync_copy(zeros_hbm, acc_sh.at[pl.ds(row0, rpt)])

    @pl.when(row0 + rpt > n)
    def _():
        pltpu.sync_copy(x_hbm.at[pl.ds(row0, short), pl.ds(cid * HALF, HALF)],
                        tbl_sh.at[pl.ds(row0, short), pl.ds(0, HALF)])
        pltpu.sync_copy(ones_hbm.at[pl.ds(0, short)],
                        tbl_sh.at[pl.ds(row0, short), pl.ds(HALF, 8)])
        pltpu.sync_copy(zeros_hbm.at[pl.ds(0, short)],
                        acc_sh.at[pl.ds(row0, short)])

    ebase = sid * ept

    def stage(s, q):
        off = ebase + s * SEGE
        pltpu.async_copy(ei_hbm.at[0, pl.ds(off, SEGE)], idx_a.at[q], sem_st)
        pltpu.async_copy(ei_hbm.at[1, pl.ds(off, SEGE)], idx_b.at[q], sem_st)

    def wait_stage():
        pltpu.make_async_copy(ei_hbm.at[0, pl.ds(0, SEGE)], idx_a.at[0],
                              sem_st).wait()
        pltpu.make_async_copy(ei_hbm.at[1, pl.ds(0, SEGE)], idx_b.at[0],
                              sem_st).wait()

    stage(0, 0)
    wait_stage()
    plsc.subcore_barrier()
    for k in range(LAG):
        start_gather(k // 2, k % 2, k % NBUF, 0)

    def seg_body(s, carry):
        q = s % 2
        qn = (s + 1) % 2
        for k in range(NSTEP):
            b = k % NBUF
            if k >= LAG:
                wait_scatter((k - LAG) % NBUF)
            else:
                @pl.when(s > 0)
                def _():
                    wait_scatter((k - LAG) % NBUF)

            if k == LAG:
                @pl.when(s + 1 < nseg)
                def _():
                    stage(s + 1, qn)

            kk = k + LAG
            if kk < NSTEP:
                start_gather(kk // 2, kk % 2, kk % NBUF, q)
            else:
                if kk == NSTEP:
                    @pl.when(s + 1 < nseg)
                    def _():
                        wait_stage()

                @pl.when(s + 1 < nseg)
                def _():
                    start_gather((kk - NSTEP) // 2, kk % 2, kk % NBUF, qn)

            wait_gather(k // 2, k % 2, b, q)
            start_scatter(k // 2, k % 2, b, q)
        return carry

    lax.fori_loop(0, nseg, seg_body, 0)
    for k in range(NSTEP - LAG, NSTEP):
        wait_scatter(k % NBUF)

    if tail:
        toff = ebase + nfull * CHUNK
        pltpu.sync_copy(ei_hbm.at[0, pl.ds(toff, tail)], tidx_a)
        pltpu.sync_copy(ei_hbm.at[1, pl.ds(toff, tail)], tidx_b)
        tidx = (tidx_a, tidx_b)
        for p in range(2):
            pltpu.async_copy(tbl_sh.at[tidx[p]],
                             rows_v.at[p, pl.ds(0, tail)], sem_g[p])
        for p in range(2):
            pltpu.make_async_copy(tbl_sh.at[tidx[p]],
                                  rows_v.at[p, pl.ds(0, tail)],
                                  sem_g[p]).wait()
            pltpu.async_copy(rows_v.at[p, pl.ds(0, tail)],
                             acc_sh.at[tidx[1 - p]], sem_s[p], add=True)
        for p in range(2):
            pltpu.make_async_copy(rows_v.at[p, pl.ds(0, tail)],
                                  acc_sh.at[tidx[1 - p]], sem_s[p]).wait()

    plsc.subcore_barrier()
    @pl.when(row0 + rpt <= n)
    def _():
        pltpu.sync_copy(acc_sh.at[pl.ds(row0, rpt)],
                        out_hbm.at[cid, pl.ds(row0, rpt)])

    @pl.when(row0 + rpt > n)
    def _():
        pltpu.sync_copy(acc_sh.at[pl.ds(row0, short)],
                        out_hbm.at[cid, pl.ds(row0, short)])


def _tc_matmul_body(h, acc_ref, x_ref, w_ref, o_ref):
    a0 = acc_ref[0]
    a1 = acc_ref[1]
    g_lo = a0[:, :HALF]
    deg = a0[:, HALF:HALF + 1]
    g_hi = a1[:, :HALF]
    o_ref[...] = (
        jnp.dot(g_lo, w_ref[:HALF], preferred_element_type=jnp.float32)
        + jnp.dot(g_hi, w_ref[HALF:h], preferred_element_type=jnp.float32)
        + jnp.dot(deg * x_ref[...], w_ref[h:], preferred_element_type=jnp.float32)
    )


def kernel(x, edge_index, W):
    n, h = x.shape
    e = edge_index.shape[1]

    rpt = (-(-n // NS) + 7) // 8 * 8
    ept = e // NS
    assert ept * NS == e and ept % 16 == 0
    tail = ept - (ept // CHUNK // SEGC) * SEGC * CHUNK
    assert 0 < tail <= 128 and tail % 8 == 0

    ei = edge_index.astype(jnp.int32)
    zeros_blk = jnp.zeros((rpt, WSC), jnp.float32)
    ones_col = jnp.ones((rpt, 8), jnp.float32)

    sc_scatter = pl.kernel(
        functools.partial(_sc_scatter_body, n, ept, rpt),
        out_type=jax.ShapeDtypeStruct((NC, n, WSC), jnp.float32),
        mesh=plsc.VectorSubcoreMesh(core_axis_name="c", subcore_axis_name="s",
                                    num_cores=NC, num_subcores=NS),
        scratch_types=[
            pltpu.VMEM((2, SEGE), jnp.int32),
            pltpu.VMEM((2, SEGE), jnp.int32),
            pltpu.VMEM((max(tail, 8),), jnp.int32),
            pltpu.VMEM((max(tail, 8),), jnp.int32),
            pltpu.VMEM((NBUF, CHUNK, WSC), jnp.float32),
            pltpu.VMEM_SHARED((n, WSC), jnp.float32),
            pltpu.VMEM_SHARED((n, WSC), jnp.float32),
            pltpu.SemaphoreType.DMA,
        ] + [pltpu.SemaphoreType.DMA] * (2 * NBUF),
        compiler_params=pltpu.CompilerParams(use_tc_tiling_on_sc=False),
    )
    acc = sc_scatter(x, ei, zeros_blk, ones_col)

    blk = 4 * rpt
    out = pl.pallas_call(
        functools.partial(_tc_matmul_body, h),
        grid=(NS // 4,),
        in_specs=[
            pl.BlockSpec((NC, blk, WSC), lambda i: (0, i, 0)),
            pl.BlockSpec((blk, h), lambda i: (i, 0)),
            pl.BlockSpec((2 * h, h), lambda i: (0, 0)),
        ],
        out_specs=pl.BlockSpec((blk, h), lambda i: (i, 0)),
        out_shape=jax.ShapeDtypeStruct((n, h), jnp.float32),
    )(acc, x, W)
    return out
